# baseline (device time: 99034 ns/iter reference)
import jax
import jax.numpy as jnp
from jax import lax
from jax.experimental import pallas as pl
from jax.experimental.pallas import tpu as pltpu

KQ = 8
HQ = 3
FWD_X = (4, 6)
FWD_Y = (3, 5, 7)
NS = KQ + len(FWD_Y)


def kernel(partial, resid, gamma):
    m, d = resid.shape
    quarter = m // 4
    mb = quarter // KQ
    x2 = partial.reshape(m, d)
    gamma2 = gamma.reshape(1, d)

    def body(p_ref, r_hbm, g_ref, out_ref, r_buf,
             copy_sems, z_send, z_recv, x_send, x_recv, y_send, y_recv):
        my_x = lax.axis_index("x")
        my_y = lax.axis_index("y")
        my_z = lax.axis_index("z")
        qb = (2 * my_x + my_y) * quarter
        xqb = (2 * (1 - my_x) + my_y) * quarter
        yqb = (2 * my_x + (1 - my_y)) * quarter
        dqb = (2 * (1 - my_x) + (1 - my_y)) * quarter
        zpeer = (my_x, my_y, 1 - my_z)
        xnbr = (1 - my_x, my_y, my_z)
        ynbr = (my_x, 1 - my_y, my_z)

        barrier_sem = pltpu.get_barrier_semaphore()
        for nbr in (zpeer, xnbr, ynbr):
            pl.semaphore_signal(
                barrier_sem, inc=1, device_id=nbr,
                device_id_type=pl.DeviceIdType.MESH)
        pl.semaphore_wait(barrier_sem, 3)

        def remote(src_rows, dst_rows, send_sem, recv_sem, dev):
            return pltpu.make_async_remote_copy(
                src_ref=out_ref.at[src_rows, :],
                dst_ref=out_ref.at[dst_rows, :],
                send_sem=send_sem, recv_sem=recv_sem,
                device_id=dev, device_id_type=pl.DeviceIdType.MESH)

        z_rdmas = []
        for i in range(KQ + HQ):
            start = qb + i * mb if i < KQ else dqb + (i - KQ) * mb
            rows = pl.ds(start, mb)
            r = pltpu.make_async_remote_copy(
                src_ref=p_ref.at[rows, :], dst_ref=out_ref.at[rows, :],
                send_sem=z_send.at[i], recv_sem=z_recv.at[i],
                device_id=zpeer, device_id_type=pl.DeviceIdType.MESH)
            r.start()
            z_rdmas.append(r)

        rc0 = pltpu.make_async_copy(
            r_hbm.at[pl.ds(qb, quarter), :],
            r_buf.at[pl.ds(0, quarter), :], copy_sems.at[0])
        rc0.start()
        rc1 = pltpu.make_async_copy(
            r_hbm.at[pl.ds(dqb, HQ * mb), :],
            r_buf.at[pl.ds(quarter, HQ * mb), :], copy_sems.at[1])
        rc1.start()
        rc0.wait()
        rc1.wait()

        def reduce_ln(rows, rbuf_off):
            y = (p_ref[rows, :] + out_ref[rows, :]
                 + r_buf[pl.ds(rbuf_off, mb), :])
            rms = jnp.sqrt(jnp.mean(y * y, axis=-1, keepdims=True) + 1e-6)
            out_ref[rows, :] = y / rms * g_ref[...]

        x_sends, y_sends = [], []
        for j in range(KQ):
            rows = pl.ds(qb + j * mb, mb)
            z_rdmas[j].wait_recv()
            reduce_ln(rows, j * mb)
            for sems, rsems, dev, lst in ((x_send, x_recv, xnbr, x_sends),
                                          (y_send, y_recv, ynbr, y_sends)):
                s = remote(rows, rows, sems.at[j], rsems.at[j], dev)
                s.start()
                lst.append(s)

        for h in range(HQ):
            rows = pl.ds(dqb + h * mb, mb)
            z_rdmas[KQ + h].wait_recv()
            reduce_ln(rows, quarter + h * mb)

        for j in range(KQ):
            xrows = pl.ds(xqb + j * mb, mb)
            yrows = pl.ds(yqb + j * mb, mb)
            remote(xrows, xrows, x_send.at[j], x_recv.at[j], xnbr).wait_recv()
            if j in FWD_Y:
                idx = KQ + FWD_Y.index(j)
                f = remote(xrows, xrows, y_send.at[idx], y_recv.at[idx], ynbr)
                f.start()
                y_sends.append(f)
            remote(yrows, yrows, y_send.at[j], y_recv.at[j], ynbr).wait_recv()
            if j in FWD_X:
                idx = KQ + FWD_X.index(j)
                f = remote(yrows, yrows, x_send.at[idx], x_recv.at[idx], xnbr)
                f.start()
                x_sends.append(f)

        for j in FWD_X:
            drows = pl.ds(dqb + j * mb, mb)
            idx = KQ + FWD_X.index(j)
            remote(drows, drows, x_send.at[idx], x_recv.at[idx],
                   xnbr).wait_recv()
        for j in FWD_Y:
            drows = pl.ds(dqb + j * mb, mb)
            idx = KQ + FWD_Y.index(j)
            remote(drows, drows, y_send.at[idx], y_recv.at[idx],
                   ynbr).wait_recv()
        for r in z_rdmas + x_sends + y_sends:
            r.wait_send()

    return pl.pallas_call(
        body,
        out_shape=jax.ShapeDtypeStruct((m, d), jnp.float32),
        in_specs=[
            pl.BlockSpec(memory_space=pltpu.VMEM),
            pl.BlockSpec(memory_space=pl.ANY),
            pl.BlockSpec(memory_space=pltpu.VMEM),
        ],
        out_specs=pl.BlockSpec(memory_space=pltpu.VMEM),
        scratch_shapes=[
            pltpu.VMEM((quarter + HQ * mb, d), jnp.float32),
            pltpu.SemaphoreType.DMA((2,)),
            pltpu.SemaphoreType.DMA((KQ + HQ,)),
            pltpu.SemaphoreType.DMA((KQ + HQ,)),
            pltpu.SemaphoreType.DMA((NS,)),
            pltpu.SemaphoreType.DMA((NS,)),
            pltpu.SemaphoreType.DMA((NS,)),
            pltpu.SemaphoreType.DMA((NS,)),
        ],
        compiler_params=pltpu.CompilerParams(collective_id=0),
    )(x2, resid, gamma2)


# device time: 87762 ns/iter; 1.1284x vs baseline; 1.1284x over previous
import jax
import jax.numpy as jnp
from jax import lax
from jax.experimental import pallas as pl
from jax.experimental.pallas import tpu as pltpu

KQ = 8
HQ = 3
FWD_X = (4, 6)
FWD_Y = (3, 5, 7)
NS = KQ + len(FWD_Y)


def kernel(partial, resid, gamma):
    m, d = resid.shape
    quarter = m // 4
    mb = quarter // KQ
    x2 = partial.reshape(m, d)
    gamma2 = gamma.reshape(1, d)

    def body(p_ref, r_hbm, g_ref, out_ref, r_buf,
             copy_sems, z_send, z_recv, x_send, x_recv, y_send, y_recv):
        my_x = lax.axis_index("x")
        my_y = lax.axis_index("y")
        my_z = lax.axis_index("z")
        qb = (2 * my_x + my_y) * quarter
        xqb = (2 * (1 - my_x) + my_y) * quarter
        yqb = (2 * my_x + (1 - my_y)) * quarter
        dqb = (2 * (1 - my_x) + (1 - my_y)) * quarter
        zpeer = (my_x, my_y, 1 - my_z)
        xnbr = (1 - my_x, my_y, my_z)
        ynbr = (my_x, 1 - my_y, my_z)

        barrier_sem = pltpu.get_barrier_semaphore()
        for nbr in (zpeer, xnbr, ynbr):
            pl.semaphore_signal(
                barrier_sem, inc=1, device_id=nbr,
                device_id_type=pl.DeviceIdType.MESH)
        pl.semaphore_wait(barrier_sem, 3)

        def remote(src_rows, dst_rows, send_sem, recv_sem, dev):
            return pltpu.make_async_remote_copy(
                src_ref=out_ref.at[src_rows, :],
                dst_ref=out_ref.at[dst_rows, :],
                send_sem=send_sem, recv_sem=recv_sem,
                device_id=dev, device_id_type=pl.DeviceIdType.MESH)

        z_rdmas = []
        for i in range(KQ + HQ):
            start = qb + i * mb if i < KQ else dqb + (i - KQ) * mb
            rows = pl.ds(start, mb)
            r = pltpu.make_async_remote_copy(
                src_ref=p_ref.at[rows, :], dst_ref=out_ref.at[rows, :],
                send_sem=z_send.at[i], recv_sem=z_recv.at[i],
                device_id=zpeer, device_id_type=pl.DeviceIdType.MESH)
            r.start()
            z_rdmas.append(r)

        rc0 = pltpu.make_async_copy(
            r_hbm.at[pl.ds(qb, quarter), :],
            r_buf.at[pl.ds(0, quarter), :], copy_sems.at[0])
        rc0.start()
        rc1 = pltpu.make_async_copy(
            r_hbm.at[pl.ds(dqb, HQ * mb), :],
            r_buf.at[pl.ds(quarter, HQ * mb), :], copy_sems.at[1])
        rc1.start()
        rc0.wait()
        rc1.wait()

        def reduce_ln(rows, rbuf_off):
            y = (p_ref[rows, :] + out_ref[rows, :]
                 + r_buf[pl.ds(rbuf_off, mb), :])
            rms = jnp.sqrt(jnp.mean(y * y, axis=-1, keepdims=True) + 1e-6)
            out_ref[rows, :] = y / rms * g_ref[...]

        x_sends, y_sends = [], []
        for j in range(KQ):
            rows = pl.ds(qb + j * mb, mb)
            z_rdmas[j].wait_recv()
            reduce_ln(rows, j * mb)
            for sems, rsems, dev, lst in ((x_send, x_recv, xnbr, x_sends),
                                          (y_send, y_recv, ynbr, y_sends)):
                s = remote(rows, rows, sems.at[j], rsems.at[j], dev)
                s.start()
                lst.append(s)

        for j in range(KQ):
            xrows = pl.ds(xqb + j * mb, mb)
            yrows = pl.ds(yqb + j * mb, mb)
            remote(xrows, xrows, x_send.at[j], x_recv.at[j], xnbr).wait_recv()
            if j in FWD_Y:
                idx = KQ + FWD_Y.index(j)
                f = remote(xrows, xrows, y_send.at[idx], y_recv.at[idx], ynbr)
                f.start()
                y_sends.append(f)
            remote(yrows, yrows, y_send.at[j], y_recv.at[j], ynbr).wait_recv()
            if j in FWD_X:
                idx = KQ + FWD_X.index(j)
                f = remote(yrows, yrows, x_send.at[idx], x_recv.at[idx], xnbr)
                f.start()
                x_sends.append(f)
            if j >= KQ - HQ:
                h = j - (KQ - HQ)
                hrows = pl.ds(dqb + h * mb, mb)
                z_rdmas[KQ + h].wait_recv()
                reduce_ln(hrows, quarter + h * mb)

        for j in FWD_X:
            drows = pl.ds(dqb + j * mb, mb)
            idx = KQ + FWD_X.index(j)
            remote(drows, drows, x_send.at[idx], x_recv.at[idx],
                   xnbr).wait_recv()
        for j in FWD_Y:
            drows = pl.ds(dqb + j * mb, mb)
            idx = KQ + FWD_Y.index(j)
            remote(drows, drows, y_send.at[idx], y_recv.at[idx],
                   ynbr).wait_recv()
        for r in z_rdmas + x_sends + y_sends:
            r.wait_send()

    return pl.pallas_call(
        body,
        out_shape=jax.ShapeDtypeStruct((m, d), jnp.float32),
        in_specs=[
            pl.BlockSpec(memory_space=pltpu.VMEM),
            pl.BlockSpec(memory_space=pl.ANY),
            pl.BlockSpec(memory_space=pltpu.VMEM),
        ],
        out_specs=pl.BlockSpec(memory_space=pltpu.VMEM),
        scratch_shapes=[
            pltpu.VMEM((quarter + HQ * mb, d), jnp.float32),
            pltpu.SemaphoreType.DMA((2,)),
            pltpu.SemaphoreType.DMA((KQ + HQ,)),
            pltpu.SemaphoreType.DMA((KQ + HQ,)),
            pltpu.SemaphoreType.DMA((NS,)),
            pltpu.SemaphoreType.DMA((NS,)),
            pltpu.SemaphoreType.DMA((NS,)),
            pltpu.SemaphoreType.DMA((NS,)),
        ],
        compiler_params=pltpu.CompilerParams(collective_id=0),
    )(x2, resid, gamma2)
